# initial kernel scaffold (unmeasured)
import jax
import jax.numpy as jnp
from jax import lax
from jax.experimental import pallas as pl
from jax.experimental.pallas import tpu as pltpu

N_DEV = 4


def kernel(x, w_mat, scale_x, scale_w):
    m_per, k = x.shape
    _, n = w_mat.shape
    n_per = n // N_DEV

    def body(x_ref, w_ref, sx_ref, sw_ref, out_ref, sendbuf, send_sems, recv_sems):
        my = lax.axis_index("i")
        s = sx_ref[0] * sw_ref[0]
        xb = x_ref[...].astype(jnp.bfloat16)

        for j in range(N_DEV):
            wj = w_ref[:, j * n_per:(j + 1) * n_per].astype(jnp.bfloat16)
            blk = jnp.maximum(
                jnp.dot(xb, wj, preferred_element_type=jnp.float32) * s, 0.0
            )

            @pl.when(my == j)
            def _():
                out_ref[j * m_per:(j + 1) * m_per, :] = blk

            @pl.when(my != j)
            def _():
                sendbuf[j] = blk
                rdma = pltpu.make_async_remote_copy(
                    src_ref=sendbuf.at[j],
                    dst_ref=out_ref.at[pl.ds(my * m_per, m_per), :],
                    send_sem=send_sems.at[j],
                    recv_sem=recv_sems.at[my],
                    device_id=(j,),
                    device_id_type=pl.DeviceIdType.MESH,
                )
                rdma.start()

        for src in range(N_DEV):

            @pl.when(my != src)
            def _():
                recv = pltpu.make_async_remote_copy(
                    src_ref=sendbuf.at[src],
                    dst_ref=out_ref.at[src * m_per:(src + 1) * m_per, :],
                    send_sem=send_sems.at[src],
                    recv_sem=recv_sems.at[src],
                    device_id=(src,),
                    device_id_type=pl.DeviceIdType.MESH,
                )
                recv.wait_recv()

        for j in range(N_DEV):

            @pl.when(my != j)
            def _():
                snd = pltpu.make_async_remote_copy(
                    src_ref=sendbuf.at[j],
                    dst_ref=out_ref.at[pl.ds(my * m_per, m_per), :],
                    send_sem=send_sems.at[j],
                    recv_sem=recv_sems.at[my],
                    device_id=(j,),
                    device_id_type=pl.DeviceIdType.MESH,
                )
                snd.wait_send()

    out_shape = jax.ShapeDtypeStruct((N_DEV * m_per, n_per), jnp.float32)
    return pl.pallas_call(
        body,
        out_shape=out_shape,
        in_specs=[
            pl.BlockSpec(memory_space=pltpu.VMEM),
            pl.BlockSpec(memory_space=pltpu.VMEM),
            pl.BlockSpec(memory_space=pltpu.SMEM),
            pl.BlockSpec(memory_space=pltpu.SMEM),
        ],
        out_specs=pl.BlockSpec(memory_space=pltpu.VMEM),
        scratch_shapes=[
            pltpu.VMEM((N_DEV, m_per, n_per), jnp.float32),
            pltpu.SemaphoreType.DMA((N_DEV,)),
            pltpu.SemaphoreType.DMA((N_DEV,)),
        ],
    )(x, w_mat, scale_x, scale_w)


# baseline (device time: 69038 ns/iter reference)
import jax
import jax.numpy as jnp
from jax import lax
from jax.experimental import pallas as pl
from jax.experimental.pallas import tpu as pltpu

N_DEV = 4
M_TILE = 256


def kernel(x, w_mat, scale_x, scale_w):
    m_per, k = x.shape
    _, n = w_mat.shape
    n_per = n // N_DEV

    def body(x_ref, w_ref, sx_ref, sw_ref, out_ref,
             sendbuf, recvbuf, send_sems, recv_sems):
        j = pl.program_id(0)
        my = lax.axis_index("i")
        s = sx_ref[0] * sw_ref[0]

        wjb = w_ref[...].astype(jnp.bfloat16)
        for mi in range(m_per // M_TILE):
            xmb = x_ref[mi * M_TILE:(mi + 1) * M_TILE, :].astype(jnp.bfloat16)
            blk = jnp.maximum(
                jnp.dot(xmb, wjb, preferred_element_type=jnp.float32) * s, 0.0
            )

            @pl.when(my == j)
            def _():
                out_ref[pl.ds(j * m_per + mi * M_TILE, M_TILE), :] = blk

            @pl.when(my != j)
            def _():
                sendbuf[j, pl.ds(mi * M_TILE, M_TILE), :] = blk.astype(jnp.bfloat16)

        @pl.when(my != j)
        def _():
            rdma = pltpu.make_async_remote_copy(
                src_ref=sendbuf.at[j],
                dst_ref=recvbuf.at[my],
                send_sem=send_sems.at[j],
                recv_sem=recv_sems.at[my],
                device_id=(j,),
                device_id_type=pl.DeviceIdType.MESH,
            )
            rdma.start()

        @pl.when(j == N_DEV - 1)
        def _():
            for src in range(N_DEV):

                @pl.when(my != src)
                def _():
                    recv = pltpu.make_async_remote_copy(
                        src_ref=sendbuf.at[src],
                        dst_ref=recvbuf.at[src],
                        send_sem=send_sems.at[src],
                        recv_sem=recv_sems.at[src],
                        device_id=(src,),
                        device_id_type=pl.DeviceIdType.MESH,
                    )
                    recv.wait_recv()
                    out_ref[src * m_per:(src + 1) * m_per, :] = (
                        recvbuf[src].astype(jnp.float32)
                    )

            for tgt in range(N_DEV):

                @pl.when(my != tgt)
                def _():
                    snd = pltpu.make_async_remote_copy(
                        src_ref=sendbuf.at[tgt],
                        dst_ref=recvbuf.at[my],
                        send_sem=send_sems.at[tgt],
                        recv_sem=recv_sems.at[my],
                        device_id=(tgt,),
                        device_id_type=pl.DeviceIdType.MESH,
                    )
                    snd.wait_send()

    out_shape = jax.ShapeDtypeStruct((N_DEV * m_per, n_per), jnp.float32)
    return pl.pallas_call(
        body,
        grid=(N_DEV,),
        out_shape=out_shape,
        in_specs=[
            pl.BlockSpec((m_per, k), lambda j: (0, 0)),
            pl.BlockSpec((k, n_per), lambda j: (0, j)),
            pl.BlockSpec(memory_space=pltpu.SMEM),
            pl.BlockSpec(memory_space=pltpu.SMEM),
        ],
        out_specs=pl.BlockSpec((N_DEV * m_per, n_per), lambda j: (0, 0)),
        scratch_shapes=[
            pltpu.VMEM((N_DEV, m_per, n_per), jnp.bfloat16),
            pltpu.VMEM((N_DEV, m_per, n_per), jnp.bfloat16),
            pltpu.SemaphoreType.DMA((N_DEV,)),
            pltpu.SemaphoreType.DMA((N_DEV,)),
        ],
        compiler_params=pltpu.CompilerParams(
            dimension_semantics=("arbitrary",),
            vmem_limit_bytes=63 * 1024 * 1024,
        ),
    )(x, w_mat, scale_x, scale_w)


# device time: 66770 ns/iter; 1.0340x vs baseline; 1.0340x over previous
import jax
import jax.numpy as jnp
from jax import lax
from jax.experimental import pallas as pl
from jax.experimental.pallas import tpu as pltpu

N_DEV = 4
M_TILE = 512


def _cast_bf16(a):
    m, k = a.shape

    def body(a_ref, o_ref):
        o_ref[...] = a_ref[...].astype(jnp.bfloat16)

    return pl.pallas_call(
        body,
        grid=(4,),
        in_specs=[pl.BlockSpec((m // 4, k), lambda i: (i, 0))],
        out_specs=pl.BlockSpec((m // 4, k), lambda i: (i, 0)),
        out_shape=jax.ShapeDtypeStruct((m, k), jnp.bfloat16),
    )(a)


def kernel(x, w_mat, scale_x, scale_w):
    m_per, k = x.shape
    _, n = w_mat.shape
    n_per = n // N_DEV

    xb = _cast_bf16(x)
    my_arr = jnp.full((1,), lax.axis_index("i"), jnp.int32)

    def body(my_ref, x_ref, w_ref, sx_ref, sw_ref, out_ref,
             sendbuf, recvbuf, send_sems, recv_sems):
        j = pl.program_id(0)
        del my_ref
        my = lax.axis_index("i")
        tgt = lax.rem(my + 1 + j, N_DEV)
        s = sx_ref[0] * sw_ref[0]

        wjb = w_ref[...].astype(jnp.bfloat16)
        for mi in range(m_per // M_TILE):
            blk = jnp.maximum(
                jnp.dot(
                    x_ref[mi * M_TILE:(mi + 1) * M_TILE, :],
                    wjb,
                    preferred_element_type=jnp.float32,
                ) * s,
                0.0,
            )

            @pl.when(tgt == my)
            def _():
                out_ref[pl.ds(my * m_per + mi * M_TILE, M_TILE), :] = blk

            @pl.when(tgt != my)
            def _():
                sendbuf[j, pl.ds(mi * M_TILE, M_TILE), :] = blk.astype(jnp.bfloat16)

        @pl.when(tgt != my)
        def _():
            rdma = pltpu.make_async_remote_copy(
                src_ref=sendbuf.at[j],
                dst_ref=recvbuf.at[my],
                send_sem=send_sems.at[j],
                recv_sem=recv_sems.at[my],
                device_id=(tgt,),
                device_id_type=pl.DeviceIdType.MESH,
            )
            rdma.start()

        @pl.when(j == N_DEV - 1)
        def _():
            for src in range(N_DEV):

                @pl.when(my != src)
                def _():
                    recv = pltpu.make_async_remote_copy(
                        src_ref=sendbuf.at[0],
                        dst_ref=recvbuf.at[src],
                        send_sem=send_sems.at[0],
                        recv_sem=recv_sems.at[src],
                        device_id=(src,),
                        device_id_type=pl.DeviceIdType.MESH,
                    )
                    recv.wait_recv()
                    out_ref[src * m_per:(src + 1) * m_per, :] = (
                        recvbuf[src].astype(jnp.float32)
                    )

            for slot in range(N_DEV - 1):
                snd = pltpu.make_async_remote_copy(
                    src_ref=sendbuf.at[slot],
                    dst_ref=recvbuf.at[my],
                    send_sem=send_sems.at[slot],
                    recv_sem=recv_sems.at[my],
                    device_id=(0,),
                    device_id_type=pl.DeviceIdType.MESH,
                )
                snd.wait_send()

    grid_spec = pltpu.PrefetchScalarGridSpec(
        num_scalar_prefetch=1,
        grid=(N_DEV,),
        in_specs=[
            pl.BlockSpec((m_per, k), lambda j, my: (0, 0)),
            pl.BlockSpec(
                (k, n_per), lambda j, my: (0, lax.rem(my[0] + 1 + j, N_DEV))
            ),
            pl.BlockSpec(memory_space=pltpu.SMEM),
            pl.BlockSpec(memory_space=pltpu.SMEM),
        ],
        out_specs=pl.BlockSpec((N_DEV * m_per, n_per), lambda j, my: (0, 0)),
        scratch_shapes=[
            pltpu.VMEM((N_DEV - 1, m_per, n_per), jnp.bfloat16),
            pltpu.VMEM((N_DEV, m_per, n_per), jnp.bfloat16),
            pltpu.SemaphoreType.DMA((N_DEV - 1,)),
            pltpu.SemaphoreType.DMA((N_DEV,)),
        ],
    )

    out_shape = jax.ShapeDtypeStruct((N_DEV * m_per, n_per), jnp.float32)
    return pl.pallas_call(
        body,
        grid_spec=grid_spec,
        out_shape=out_shape,
        compiler_params=pltpu.CompilerParams(
            dimension_semantics=("arbitrary",),
            vmem_limit_bytes=63 * 1024 * 1024,
        ),
    )(my_arr, xb, w_mat, scale_x, scale_w)


# device time: 59059 ns/iter; 1.1690x vs baseline; 1.1306x over previous
import jax
import jax.numpy as jnp
from jax import lax
from jax.experimental import pallas as pl
from jax.experimental.pallas import tpu as pltpu

N_DEV = 4
X_DT = jnp.float8_e4m3fn
W_DT = jnp.float8_e5m2


def _cast_x(a):
    m, k = a.shape

    def body(a_ref, o_ref):
        o_ref[...] = a_ref[...].astype(X_DT)

    return pl.pallas_call(
        body,
        grid=(4,),
        in_specs=[pl.BlockSpec((m // 4, k), lambda i: (i, 0))],
        out_specs=pl.BlockSpec((m // 4, k), lambda i: (i, 0)),
        out_shape=jax.ShapeDtypeStruct((m, k), X_DT),
    )(a)


def kernel(x, w_mat, scale_x, scale_w):
    m_per, k = x.shape
    _, n = w_mat.shape
    n_per = n // N_DEV

    xq = _cast_x(x)
    my_arr = jnp.full((1,), lax.axis_index("i"), jnp.int32)

    def body(my_ref, x_ref, w_ref, sx_ref, sw_ref, out_ref,
             sendbuf, recvbuf, send_sems, recv_sems):
        j = pl.program_id(0)
        del my_ref
        my = lax.axis_index("i")
        tgt = lax.rem(my + 1 + j, N_DEV)
        s = sx_ref[0] * sw_ref[0]

        wq = w_ref[...].astype(W_DT)
        blk = jnp.maximum(
            jnp.dot(x_ref[...], wq, preferred_element_type=jnp.float32) * s,
            0.0,
        )

        @pl.when(j == N_DEV - 1)
        def _():
            out_ref[pl.ds(my * m_per, m_per), :] = blk

        @pl.when(j < N_DEV - 1)
        def _():
            sendbuf[j] = blk.astype(jnp.bfloat16)
            rdma = pltpu.make_async_remote_copy(
                src_ref=sendbuf.at[j],
                dst_ref=recvbuf.at[my],
                send_sem=send_sems.at[j],
                recv_sem=recv_sems.at[my],
                device_id=(tgt,),
                device_id_type=pl.DeviceIdType.MESH,
            )
            rdma.start()

        def wait_and_store(d):
            src = lax.rem(my + N_DEV - d, N_DEV)
            recv = pltpu.make_async_remote_copy(
                src_ref=sendbuf.at[0],
                dst_ref=recvbuf.at[src],
                send_sem=send_sems.at[0],
                recv_sem=recv_sems.at[src],
                device_id=(0,),
                device_id_type=pl.DeviceIdType.MESH,
            )
            recv.wait_recv()
            out_ref[pl.ds(src * m_per, m_per), :] = recvbuf[src].astype(
                jnp.float32
            )

        @pl.when(j == N_DEV - 2)
        def _():
            wait_and_store(1)

        @pl.when(j == N_DEV - 1)
        def _():
            wait_and_store(2)
            wait_and_store(3)
            for slot in range(N_DEV - 1):
                snd = pltpu.make_async_remote_copy(
                    src_ref=sendbuf.at[slot],
                    dst_ref=recvbuf.at[my],
                    send_sem=send_sems.at[slot],
                    recv_sem=recv_sems.at[my],
                    device_id=(0,),
                    device_id_type=pl.DeviceIdType.MESH,
                )
                snd.wait_send()

    grid_spec = pltpu.PrefetchScalarGridSpec(
        num_scalar_prefetch=1,
        grid=(N_DEV,),
        in_specs=[
            pl.BlockSpec((m_per, k), lambda j, my: (0, 0)),
            pl.BlockSpec(
                (k, n_per), lambda j, my: (0, lax.rem(my[0] + 1 + j, N_DEV))
            ),
            pl.BlockSpec(memory_space=pltpu.SMEM),
            pl.BlockSpec(memory_space=pltpu.SMEM),
        ],
        out_specs=pl.BlockSpec((N_DEV * m_per, n_per), lambda j, my: (0, 0)),
        scratch_shapes=[
            pltpu.VMEM((N_DEV - 1, m_per, n_per), jnp.bfloat16),
            pltpu.VMEM((N_DEV, m_per, n_per), jnp.bfloat16),
            pltpu.SemaphoreType.DMA((N_DEV - 1,)),
            pltpu.SemaphoreType.DMA((N_DEV,)),
        ],
    )

    out_shape = jax.ShapeDtypeStruct((N_DEV * m_per, n_per), jnp.float32)
    return pl.pallas_call(
        body,
        grid_spec=grid_spec,
        out_shape=out_shape,
        compiler_params=pltpu.CompilerParams(
            dimension_semantics=("arbitrary",),
            vmem_limit_bytes=63 * 1024 * 1024,
        ),
    )(my_arr, xq, w_mat, scale_x, scale_w)
